# baseline (speedup 1.0000x reference)
import functools

import jax
import jax.numpy as jnp
from jax import lax
from jax.experimental import pallas as pl
from jax.experimental.pallas import tpu as pltpu
from jax.experimental.pallas import tpu_sc as plsc

N_NODES = 50000
D_FEAT = 512
NUM_SEGMENTS = 1024
EPSILON = 1e-06

_L = 16
_DV = D_FEAT // _L
_NW = 32

_NH = 2
_NG = 16
_GW = 2 * _L
_HROWS = N_NODES // _NH
_C1 = 250
_NCH1 = _HROWS // _C1
_UNROLL = 5
_STRIDE = _C1 // _UNROLL

_C3 = 40
_NCH3 = N_NODES // _C3
_MAXI3 = -(-_NCH3 // _NW)

_params = pltpu.CompilerParams(use_tc_tiling_on_sc=False,
                               needs_layout_passes=False)
_mesh = plsc.VectorSubcoreMesh(core_axis_name="c", subcore_axis_name="s")


@functools.partial(
    pl.kernel,
    out_type=(
        jax.ShapeDtypeStruct((_NH * NUM_SEGMENTS, D_FEAT), jnp.float32),
        jax.ShapeDtypeStruct((_NH * NUM_SEGMENTS, D_FEAT), jnp.float32),
        jax.ShapeDtypeStruct((_NW * NUM_SEGMENTS, _L), jnp.float32),
    ),
    mesh=_mesh,
    scratch_types=[
        pltpu.VMEM((_HROWS,), jnp.int32),
        pltpu.VMEM((_C1, _GW), jnp.float32),
        pltpu.VMEM((_C1, _GW), jnp.float32),
        pltpu.VMEM((NUM_SEGMENTS, _GW), jnp.float32),
        pltpu.VMEM((NUM_SEGMENTS, _GW), jnp.float32),
        pltpu.VMEM((NUM_SEGMENTS, _L), jnp.float32),
        pltpu.SemaphoreType.DMA,
        pltpu.SemaphoreType.DMA,
    ],
    compiler_params=_params,
)
def _k_reduce(x_hbm, ids_hbm, s1_out, s2_out, cnt_out,
              iv, xb0, xb1, acc1, acc2, accc, sem0, sem1):
    cid = lax.axis_index("c")
    sid = lax.axis_index("s")
    w = sid * 2 + cid
    h = w // _NG
    g = lax.rem(w, _NG)
    f0 = g * _GW
    row0 = h * _HROWS

    zeros16 = jnp.zeros((_L,), jnp.float32)
    ones16 = jnp.ones((_L,), jnp.float32)
    cols = lax.iota(jnp.int32, _L)
    cols2 = cols + _L

    idcp = pltpu.async_copy(ids_hbm.at[pl.ds(row0, _HROWS)], iv, sem0)

    def _init(r, carry):
        acc1[r, pl.ds(0, _L)] = zeros16
        acc1[r, pl.ds(_L, _L)] = zeros16
        acc2[r, pl.ds(0, _L)] = zeros16
        acc2[r, pl.ds(_L, _L)] = zeros16
        accc[r, :] = zeros16
        return carry
    lax.fori_loop(0, NUM_SEGMENTS, _init, 0)
    idcp.wait()

    bufs = (xb0, xb1)
    sems = (sem0, sem1)

    def _issue(k, b):
        pltpu.async_copy(
            x_hbm.at[pl.ds(row0 + k * _C1, _C1), pl.ds(f0, _GW)],
            bufs[b], sems[b])

    def _wait(b):
        pltpu.make_async_copy(
            x_hbm.at[pl.ds(0, _C1), pl.ds(0, _GW)], bufs[b], sems[b]).wait()

    def _process(k, b):
        xb = bufs[b]
        rbase = k * _C1

        def _rows(jj, carry):
            for p in range(_UNROLL):
                r = p * _STRIDE + jj
                seg = plsc.load_gather(
                    iv, [jnp.full((_L,), rbase + r, jnp.int32)])
                v0 = xb[r, pl.ds(0, _L)]
                v1 = xb[r, pl.ds(_L, _L)]
                plsc.addupdate_scatter(acc1, [seg, cols], v0)
                plsc.addupdate_scatter(acc1, [seg, cols2], v1)
                plsc.addupdate_scatter(acc2, [seg, cols], v0 * v0)
                plsc.addupdate_scatter(acc2, [seg, cols2], v1 * v1)
            return carry
        lax.fori_loop(0, _STRIDE, _rows, 0)

        @pl.when(lax.rem(k, _NG) == g)
        def _():
            def _crows(jj, carry):
                for p in range(_UNROLL):
                    r = p * _STRIDE + jj
                    seg = plsc.load_gather(
                        iv, [jnp.full((_L,), rbase + r, jnp.int32)])
                    plsc.addupdate_scatter(accc, [seg, cols], ones16)
                return carry
            lax.fori_loop(0, _STRIDE, _crows, 0)

    _issue(0, 0)
    _issue(1, 1)

    def _outer(ko, carry):
        k = ko * 2
        for b in range(2):
            kk = k + b
            _wait(b)
            _process(kk, b)

            @pl.when(kk + 2 < _NCH1)
            def _():
                _issue(kk + 2, b)
        return carry
    lax.fori_loop(0, _NCH1 // 2, _outer, 0)

    obase = h * NUM_SEGMENTS
    pltpu.sync_copy(
        acc1, s1_out.at[pl.ds(obase, NUM_SEGMENTS), pl.ds(f0, _GW)])
    pltpu.sync_copy(
        acc2, s2_out.at[pl.ds(obase, NUM_SEGMENTS), pl.ds(f0, _GW)])
    pltpu.sync_copy(accc, cnt_out.at[pl.ds(w * NUM_SEGMENTS, NUM_SEGMENTS)])


def _k_table_body(s1_ref, s2_ref, cnt_ref, bias_ref, t_ref):
    s1 = s1_ref[: NUM_SEGMENTS, :] + s1_ref[NUM_SEGMENTS:, :]
    s2 = s2_ref[: NUM_SEGMENTS, :] + s2_ref[NUM_SEGMENTS:, :]
    cnt = jnp.sum(
        cnt_ref[:, 0].reshape(_NW, NUM_SEGMENTS), axis=0)
    c = jnp.maximum(cnt, 1.0)[:, None]
    mean = s1 / c
    b = bias_ref[0]
    var = s2 / c - mean * mean + b * b
    a = lax.rsqrt(var + EPSILON)
    t_ref[:, :D_FEAT] = a
    t_ref[:, D_FEAT:] = (b - mean) * a


_WIN = 16


@functools.partial(
    pl.kernel,
    out_type=jax.ShapeDtypeStruct((N_NODES, D_FEAT), jnp.float32),
    mesh=_mesh,
    scratch_types=[
        pltpu.VMEM((_C3, D_FEAT), jnp.float32),
        pltpu.VMEM((_C3, D_FEAT), jnp.float32),
        pltpu.VMEM((_WIN, 2 * D_FEAT), jnp.float32),
        pltpu.VMEM((_WIN, 2 * D_FEAT), jnp.float32),
        pltpu.VMEM((1, 2 * D_FEAT), jnp.float32),
        pltpu.VMEM((_MAXI3 * _C3,), jnp.int32),
        pltpu.SemaphoreType.DMA,
        pltpu.SemaphoreType.DMA,
        pltpu.SemaphoreType.DMA,
        pltpu.SemaphoreType.DMA,
        pltpu.SemaphoreType.DMA,
    ],
    compiler_params=_params,
)
def _k_apply(x_hbm, ids_hbm, t_hbm, out_hbm,
             xb0, xb1, tw0, tw1, trow, ivall,
             gsem0, gsem1, osem0, osem1, isem):
    cid = lax.axis_index("c")
    sid = lax.axis_index("s")
    wid = sid * 2 + cid

    xbufs = (xb0, xb1)
    twins = (tw0, tw1)
    gsems = (gsem0, gsem1)
    osems = (osem0, osem1)
    cols = lax.iota(jnp.int32, _L)

    def _seg_scalar(j):
        return lax.reduce_min(
            plsc.load_gather(ivall, [jnp.full((_L,), j, jnp.int32)]), (0,))

    def _win_base(i):
        return jnp.minimum(_seg_scalar(i * _C3),
                           jnp.int32(NUM_SEGMENTS - _WIN))

    for j in range(_MAXI3):
        kj = wid + j * _NW

        @pl.when(kj < _NCH3)
        def _():
            pltpu.async_copy(ids_hbm.at[pl.ds(kj * _C3, _C3)],
                             ivall.at[pl.ds(j * _C3, _C3)], isem)
    for j in range(_MAXI3):
        kj = wid + j * _NW

        @pl.when(kj < _NCH3)
        def _():
            pltpu.make_async_copy(ids_hbm.at[pl.ds(0, _C3)],
                                  ivall.at[pl.ds(0, _C3)], isem).wait()

    def _issue_pre_b(i, b):
        k = wid + i * _NW
        pltpu.async_copy(t_hbm.at[pl.ds(_win_base(i), _WIN)],
                         twins[b], gsems[b])
        pltpu.async_copy(x_hbm.at[pl.ds(k * _C3, _C3)], xbufs[b], gsems[b])

    def _wait_pre(b):
        pltpu.make_async_copy(t_hbm.at[pl.ds(0, _WIN)],
                              twins[b], gsems[b]).wait()
        pltpu.make_async_copy(x_hbm.at[pl.ds(0, _C3)],
                              xbufs[b], gsems[b]).wait()

    def _wait_out(b):
        pltpu.make_async_copy(
            xbufs[b], out_hbm.at[pl.ds(0, _C3)], osems[b]).wait()

    _issue_pre_b(0, 0)

    def _outer(io, carry):
        i0 = io * 2
        for b in range(2):
            i = i0 + b
            k = wid + i * _NW

            @pl.when(k < _NCH3)
            def _():
                @pl.when(i >= 1)
                def _():
                    _wait_out(1 - b)

                @pl.when(k + _NW < _NCH3)
                def _():
                    _issue_pre_b(i + 1, 1 - b)

                _wait_pre(b)

                xb = xbufs[b]
                tw = twins[b]
                s0 = _win_base(i)
                slast = _seg_scalar(i * _C3 + _C3 - 1)

                @pl.when(slast < s0 + _WIN)
                def _():
                    s0v = jnp.full((_L,), s0, jnp.int32)

                    def _row(r, carry2):
                        lseg = plsc.load_gather(
                            ivall,
                            [jnp.full((_L,), i * _C3 + r, jnp.int32)]) - s0v

                        def _col(c, carry3):
                            cv = cols + c * _L
                            a = plsc.load_gather(tw, [lseg, cv])
                            bv = plsc.load_gather(tw, [lseg, cv + D_FEAT])
                            v = xb[r, pl.ds(c * _L, _L)]
                            xb[r, pl.ds(c * _L, _L)] = v * a + bv
                            return carry3
                        return lax.fori_loop(0, _DV, _col, carry2)
                    lax.fori_loop(0, _C3, _row, 0)

                @pl.when(slast >= s0 + _WIN)
                def _():
                    def _row(r, carry2):
                        sr = _seg_scalar(i * _C3 + r)
                        pltpu.sync_copy(t_hbm.at[pl.ds(sr, 1)], trow)

                        def _col(c, carry3):
                            a = trow[0, pl.ds(c * _L, _L)]
                            bv = trow[0, pl.ds(D_FEAT + c * _L, _L)]
                            v = xb[r, pl.ds(c * _L, _L)]
                            xb[r, pl.ds(c * _L, _L)] = v * a + bv
                            return carry3
                        return lax.fori_loop(0, _DV, _col, carry2)
                    lax.fori_loop(0, _C3, _row, 0)

                pltpu.async_copy(xb, out_hbm.at[pl.ds(k * _C3, _C3)],
                                 osems[b])
        return carry
    lax.fori_loop(0, _MAXI3 // 2, _outer, 0)

    @pl.when(wid < _NCH3 - (_MAXI3 - 1) * _NW)
    def _():
        _wait_out(1)

    @pl.when(wid >= _NCH3 - (_MAXI3 - 1) * _NW)
    def _():
        _wait_out(0)


def kernel(inputs, graph_mask, bias):
    ids = graph_mask.astype(jnp.int32)
    s1, s2, cnt = _k_reduce(inputs, ids)
    table = pl.pallas_call(
        _k_table_body,
        out_shape=jax.ShapeDtypeStruct((NUM_SEGMENTS, 2 * D_FEAT), jnp.float32),
    )(s1, s2, cnt, bias.reshape(1, D_FEAT))
    return _k_apply(inputs, ids, table)


# speedup vs baseline: 1.0630x; 1.0630x over previous
import functools

import jax
import jax.numpy as jnp
from jax import lax
from jax.experimental import pallas as pl
from jax.experimental.pallas import tpu as pltpu
from jax.experimental.pallas import tpu_sc as plsc

N_NODES = 50000
D_FEAT = 512
NUM_SEGMENTS = 1024
EPSILON = 1e-06

_L = 16
_DV = D_FEAT // _L
_NW = 32

_NH = 2
_NG = 16
_GW = 2 * _L
_HROWS = N_NODES // _NH
_C1 = 250
_NCH1 = _HROWS // _C1
_UNROLL = 10
_STRIDE = _C1 // _UNROLL

_C3 = 40
_NCH3 = N_NODES // _C3
_MAXI3 = -(-_NCH3 // _NW)

_params = pltpu.CompilerParams(use_tc_tiling_on_sc=False,
                               needs_layout_passes=False)
_mesh = plsc.VectorSubcoreMesh(core_axis_name="c", subcore_axis_name="s")


@functools.partial(
    pl.kernel,
    out_type=(
        jax.ShapeDtypeStruct((_NH * NUM_SEGMENTS, D_FEAT), jnp.float32),
        jax.ShapeDtypeStruct((_NH * NUM_SEGMENTS, D_FEAT), jnp.float32),
        jax.ShapeDtypeStruct((_NW * NUM_SEGMENTS, _L), jnp.float32),
    ),
    mesh=_mesh,
    scratch_types=[
        pltpu.VMEM((_HROWS,), jnp.int32),
        pltpu.VMEM((_C1, _GW), jnp.float32),
        pltpu.VMEM((_C1, _GW), jnp.float32),
        pltpu.VMEM((NUM_SEGMENTS, _GW), jnp.float32),
        pltpu.VMEM((NUM_SEGMENTS, _GW), jnp.float32),
        pltpu.VMEM((NUM_SEGMENTS, _L), jnp.float32),
        pltpu.SemaphoreType.DMA,
        pltpu.SemaphoreType.DMA,
    ],
    compiler_params=_params,
)
def _k_reduce(x_hbm, ids_hbm, s1_out, s2_out, cnt_out,
              iv, xb0, xb1, acc1, acc2, accc, sem0, sem1):
    cid = lax.axis_index("c")
    sid = lax.axis_index("s")
    w = sid * 2 + cid
    h = w // _NG
    g = lax.rem(w, _NG)
    f0 = g * _GW
    row0 = h * _HROWS

    zeros16 = jnp.zeros((_L,), jnp.float32)
    ones16 = jnp.ones((_L,), jnp.float32)
    cols = lax.iota(jnp.int32, _L)
    cols2 = cols + _L

    idcp = pltpu.async_copy(ids_hbm.at[pl.ds(row0, _HROWS)], iv, sem0)

    def _init(r, carry):
        acc1[r, pl.ds(0, _L)] = zeros16
        acc1[r, pl.ds(_L, _L)] = zeros16
        acc2[r, pl.ds(0, _L)] = zeros16
        acc2[r, pl.ds(_L, _L)] = zeros16
        accc[r, :] = zeros16
        return carry
    lax.fori_loop(0, NUM_SEGMENTS, _init, 0)
    idcp.wait()

    bufs = (xb0, xb1)
    sems = (sem0, sem1)

    def _issue(k, b):
        pltpu.async_copy(
            x_hbm.at[pl.ds(row0 + k * _C1, _C1), pl.ds(f0, _GW)],
            bufs[b], sems[b])

    def _wait(b):
        pltpu.make_async_copy(
            x_hbm.at[pl.ds(0, _C1), pl.ds(0, _GW)], bufs[b], sems[b]).wait()

    def _process(k, b):
        xb = bufs[b]
        rbase = k * _C1

        def _rows(jj, carry):
            for p in range(_UNROLL):
                r = p * _STRIDE + jj
                seg = plsc.load_gather(
                    iv, [jnp.full((_L,), rbase + r, jnp.int32)])
                v0 = xb[r, pl.ds(0, _L)]
                v1 = xb[r, pl.ds(_L, _L)]
                plsc.addupdate_scatter(acc1, [seg, cols], v0)
                plsc.addupdate_scatter(acc1, [seg, cols2], v1)
                plsc.addupdate_scatter(acc2, [seg, cols], v0 * v0)
                plsc.addupdate_scatter(acc2, [seg, cols2], v1 * v1)
            return carry
        lax.fori_loop(0, _STRIDE, _rows, 0)

        @pl.when(lax.rem(k, _NG) == g)
        def _():
            def _crows(jj, carry):
                for p in range(_UNROLL):
                    r = p * _STRIDE + jj
                    seg = plsc.load_gather(
                        iv, [jnp.full((_L,), rbase + r, jnp.int32)])
                    plsc.addupdate_scatter(accc, [seg, cols], ones16)
                return carry
            lax.fori_loop(0, _STRIDE, _crows, 0)

    _issue(0, 0)
    _issue(1, 1)

    def _outer(ko, carry):
        k = ko * 2
        for b in range(2):
            kk = k + b
            _wait(b)
            _process(kk, b)

            @pl.when(kk + 2 < _NCH1)
            def _():
                _issue(kk + 2, b)
        return carry
    lax.fori_loop(0, _NCH1 // 2, _outer, 0)

    obase = h * NUM_SEGMENTS
    pltpu.sync_copy(
        acc1, s1_out.at[pl.ds(obase, NUM_SEGMENTS), pl.ds(f0, _GW)])
    pltpu.sync_copy(
        acc2, s2_out.at[pl.ds(obase, NUM_SEGMENTS), pl.ds(f0, _GW)])
    pltpu.sync_copy(accc, cnt_out.at[pl.ds(w * NUM_SEGMENTS, NUM_SEGMENTS)])


def _k_table_body(s1_ref, s2_ref, cnt_ref, bias_ref, t_ref):
    s1 = s1_ref[: NUM_SEGMENTS, :] + s1_ref[NUM_SEGMENTS:, :]
    s2 = s2_ref[: NUM_SEGMENTS, :] + s2_ref[NUM_SEGMENTS:, :]
    cnt = jnp.sum(
        cnt_ref[:, 0].reshape(_NW, NUM_SEGMENTS), axis=0)
    c = jnp.maximum(cnt, 1.0)[:, None]
    mean = s1 / c
    b = bias_ref[0]
    var = s2 / c - mean * mean + b * b
    a = lax.rsqrt(var + EPSILON)
    t_ref[:, :D_FEAT] = a
    t_ref[:, D_FEAT:] = (b - mean) * a


_WIN = 16


@functools.partial(
    pl.kernel,
    out_type=jax.ShapeDtypeStruct((N_NODES, D_FEAT), jnp.float32),
    mesh=_mesh,
    scratch_types=[
        pltpu.VMEM((_C3, D_FEAT), jnp.float32),
        pltpu.VMEM((_C3, D_FEAT), jnp.float32),
        pltpu.VMEM((_WIN, 2 * D_FEAT), jnp.float32),
        pltpu.VMEM((_WIN, 2 * D_FEAT), jnp.float32),
        pltpu.VMEM((1, 2 * D_FEAT), jnp.float32),
        pltpu.VMEM((_MAXI3 * _C3,), jnp.int32),
        pltpu.SemaphoreType.DMA,
        pltpu.SemaphoreType.DMA,
        pltpu.SemaphoreType.DMA,
        pltpu.SemaphoreType.DMA,
        pltpu.SemaphoreType.DMA,
    ],
    compiler_params=_params,
)
def _k_apply(x_hbm, ids_hbm, t_hbm, out_hbm,
             xb0, xb1, tw0, tw1, trow, ivall,
             gsem0, gsem1, osem0, osem1, isem):
    cid = lax.axis_index("c")
    sid = lax.axis_index("s")
    wid = sid * 2 + cid

    xbufs = (xb0, xb1)
    twins = (tw0, tw1)
    gsems = (gsem0, gsem1)
    osems = (osem0, osem1)
    cols = lax.iota(jnp.int32, _L)

    def _seg_scalar(j):
        return lax.reduce_min(
            plsc.load_gather(ivall, [jnp.full((_L,), j, jnp.int32)]), (0,))

    def _win_base(i):
        return jnp.minimum(_seg_scalar(i * _C3),
                           jnp.int32(NUM_SEGMENTS - _WIN))

    for j in range(_MAXI3):
        kj = wid + j * _NW

        @pl.when(kj < _NCH3)
        def _():
            pltpu.async_copy(ids_hbm.at[pl.ds(kj * _C3, _C3)],
                             ivall.at[pl.ds(j * _C3, _C3)], isem)
    for j in range(_MAXI3):
        kj = wid + j * _NW

        @pl.when(kj < _NCH3)
        def _():
            pltpu.make_async_copy(ids_hbm.at[pl.ds(0, _C3)],
                                  ivall.at[pl.ds(0, _C3)], isem).wait()

    def _issue_pre_b(i, b):
        k = wid + i * _NW
        pltpu.async_copy(t_hbm.at[pl.ds(_win_base(i), _WIN)],
                         twins[b], gsems[b])
        pltpu.async_copy(x_hbm.at[pl.ds(k * _C3, _C3)], xbufs[b], gsems[b])

    def _wait_pre(b):
        pltpu.make_async_copy(t_hbm.at[pl.ds(0, _WIN)],
                              twins[b], gsems[b]).wait()
        pltpu.make_async_copy(x_hbm.at[pl.ds(0, _C3)],
                              xbufs[b], gsems[b]).wait()

    def _wait_out(b):
        pltpu.make_async_copy(
            xbufs[b], out_hbm.at[pl.ds(0, _C3)], osems[b]).wait()

    _issue_pre_b(0, 0)

    def _outer(io, carry):
        i0 = io * 2
        for b in range(2):
            i = i0 + b
            k = wid + i * _NW

            @pl.when(k < _NCH3)
            def _():
                @pl.when(i >= 1)
                def _():
                    _wait_out(1 - b)

                @pl.when(k + _NW < _NCH3)
                def _():
                    _issue_pre_b(i + 1, 1 - b)

                _wait_pre(b)

                xb = xbufs[b]
                tw = twins[b]
                s0 = _win_base(i)
                slast = _seg_scalar(i * _C3 + _C3 - 1)

                @pl.when(slast < s0 + _WIN)
                def _():
                    s0v = jnp.full((_L,), s0, jnp.int32)

                    def _row(r, carry2):
                        lseg = plsc.load_gather(
                            ivall,
                            [jnp.full((_L,), i * _C3 + r, jnp.int32)]) - s0v
                        for c in range(_DV):
                            cv = cols + c * _L
                            a = plsc.load_gather(tw, [lseg, cv])
                            bv = plsc.load_gather(tw, [lseg, cv + D_FEAT])
                            v = xb[r, pl.ds(c * _L, _L)]
                            xb[r, pl.ds(c * _L, _L)] = v * a + bv
                        return carry2
                    lax.fori_loop(0, _C3, _row, 0)

                @pl.when(slast >= s0 + _WIN)
                def _():
                    def _row(r, carry2):
                        sr = _seg_scalar(i * _C3 + r)
                        pltpu.sync_copy(t_hbm.at[pl.ds(sr, 1)], trow)
                        for c in range(_DV):
                            a = trow[0, pl.ds(c * _L, _L)]
                            bv = trow[0, pl.ds(D_FEAT + c * _L, _L)]
                            v = xb[r, pl.ds(c * _L, _L)]
                            xb[r, pl.ds(c * _L, _L)] = v * a + bv
                        return carry2
                    lax.fori_loop(0, _C3, _row, 0)

                pltpu.async_copy(xb, out_hbm.at[pl.ds(k * _C3, _C3)],
                                 osems[b])
        return carry
    lax.fori_loop(0, _MAXI3 // 2, _outer, 0)

    @pl.when(wid < _NCH3 - (_MAXI3 - 1) * _NW)
    def _():
        _wait_out(1)

    @pl.when(wid >= _NCH3 - (_MAXI3 - 1) * _NW)
    def _():
        _wait_out(0)


def kernel(inputs, graph_mask, bias):
    ids = graph_mask.astype(jnp.int32)
    s1, s2, cnt = _k_reduce(inputs, ids)
    table = pl.pallas_call(
        _k_table_body,
        out_shape=jax.ShapeDtypeStruct((NUM_SEGMENTS, 2 * D_FEAT), jnp.float32),
    )(s1, s2, cnt, bias.reshape(1, D_FEAT))
    return _k_apply(inputs, ids, table)


# speedup vs baseline: 1.0633x; 1.0003x over previous
import functools

import jax
import jax.numpy as jnp
from jax import lax
from jax.experimental import pallas as pl
from jax.experimental.pallas import tpu as pltpu
from jax.experimental.pallas import tpu_sc as plsc

N_NODES = 50000
D_FEAT = 512
NUM_SEGMENTS = 1024
EPSILON = 1e-06

_L = 16
_DV = D_FEAT // _L
_NW = 32

_NH = 2
_NG = 16
_GW = 2 * _L
_HROWS = N_NODES // _NH
_C1 = 250
_NCH1 = _HROWS // _C1
_UNROLL = 10
_STRIDE = _C1 // _UNROLL

_C3 = 40
_NCH3 = N_NODES // _C3
_MAXI3 = -(-_NCH3 // _NW)

_params = pltpu.CompilerParams(use_tc_tiling_on_sc=False,
                               needs_layout_passes=False)
_mesh = plsc.VectorSubcoreMesh(core_axis_name="c", subcore_axis_name="s")


@functools.partial(
    pl.kernel,
    out_type=(
        jax.ShapeDtypeStruct((_NH * NUM_SEGMENTS, D_FEAT), jnp.float32),
        jax.ShapeDtypeStruct((_NH * NUM_SEGMENTS, D_FEAT), jnp.float32),
        jax.ShapeDtypeStruct((_NW * NUM_SEGMENTS, _L), jnp.float32),
    ),
    mesh=_mesh,
    scratch_types=[
        pltpu.VMEM((_HROWS,), jnp.int32),
        pltpu.VMEM((_C1, _GW), jnp.float32),
        pltpu.VMEM((_C1, _GW), jnp.float32),
        pltpu.VMEM((NUM_SEGMENTS, _GW), jnp.float32),
        pltpu.VMEM((NUM_SEGMENTS, _GW), jnp.float32),
        pltpu.VMEM((NUM_SEGMENTS, _L), jnp.float32),
        pltpu.SemaphoreType.DMA,
        pltpu.SemaphoreType.DMA,
    ],
    compiler_params=_params,
)
def _k_reduce(x_hbm, ids_hbm, s1_out, s2_out, cnt_out,
              iv, xb0, xb1, acc1, acc2, accc, sem0, sem1):
    cid = lax.axis_index("c")
    sid = lax.axis_index("s")
    w = sid * 2 + cid
    h = w // _NG
    g = lax.rem(w, _NG)
    f0 = g * _GW
    row0 = h * _HROWS

    zeros16 = jnp.zeros((_L,), jnp.float32)
    ones16 = jnp.ones((_L,), jnp.float32)
    cols = lax.iota(jnp.int32, _L)
    cols2 = cols + _L

    idcp = pltpu.async_copy(ids_hbm.at[pl.ds(row0, _HROWS)], iv, sem0)

    def _init(r, carry):
        acc1[r, pl.ds(0, _L)] = zeros16
        acc1[r, pl.ds(_L, _L)] = zeros16
        acc2[r, pl.ds(0, _L)] = zeros16
        acc2[r, pl.ds(_L, _L)] = zeros16
        accc[r, :] = zeros16
        return carry
    lax.fori_loop(0, NUM_SEGMENTS, _init, 0)
    idcp.wait()

    bufs = (xb0, xb1)
    sems = (sem0, sem1)

    def _issue(k, b):
        pltpu.async_copy(
            x_hbm.at[pl.ds(row0 + k * _C1, _C1), pl.ds(f0, _GW)],
            bufs[b], sems[b])

    def _wait(b):
        pltpu.make_async_copy(
            x_hbm.at[pl.ds(0, _C1), pl.ds(0, _GW)], bufs[b], sems[b]).wait()

    def _process(k, b):
        xb = bufs[b]
        rbase = k * _C1

        def _rows(jj, carry):
            for p in range(_UNROLL):
                r = p * _STRIDE + jj
                seg = plsc.load_gather(
                    iv, [jnp.full((_L,), rbase + r, jnp.int32)])
                v0 = xb[r, pl.ds(0, _L)]
                v1 = xb[r, pl.ds(_L, _L)]
                plsc.addupdate_scatter(acc1, [seg, cols], v0)
                plsc.addupdate_scatter(acc1, [seg, cols2], v1)
                plsc.addupdate_scatter(acc2, [seg, cols], v0 * v0)
                plsc.addupdate_scatter(acc2, [seg, cols2], v1 * v1)
            return carry
        lax.fori_loop(0, _STRIDE, _rows, 0)

        @pl.when(lax.rem(k, _NG) == g)
        def _():
            def _crows(jj, carry):
                for p in range(_UNROLL):
                    r = p * _STRIDE + jj
                    seg = plsc.load_gather(
                        iv, [jnp.full((_L,), rbase + r, jnp.int32)])
                    plsc.addupdate_scatter(accc, [seg, cols], ones16)
                return carry
            lax.fori_loop(0, _STRIDE, _crows, 0)

    _issue(0, 0)
    _issue(1, 1)

    def _outer(ko, carry):
        k = ko * 2
        for b in range(2):
            kk = k + b
            _wait(b)
            _process(kk, b)

            @pl.when(kk + 2 < _NCH1)
            def _():
                _issue(kk + 2, b)
        return carry
    lax.fori_loop(0, _NCH1 // 2, _outer, 0)

    obase = h * NUM_SEGMENTS
    pltpu.sync_copy(
        acc1, s1_out.at[pl.ds(obase, NUM_SEGMENTS), pl.ds(f0, _GW)])
    pltpu.sync_copy(
        acc2, s2_out.at[pl.ds(obase, NUM_SEGMENTS), pl.ds(f0, _GW)])
    pltpu.sync_copy(accc, cnt_out.at[pl.ds(w * NUM_SEGMENTS, NUM_SEGMENTS)])


def _k_table_body(s1_ref, s2_ref, cnt_ref, bias_ref, t_ref):
    s1 = s1_ref[: NUM_SEGMENTS, :] + s1_ref[NUM_SEGMENTS:, :]
    s2 = s2_ref[: NUM_SEGMENTS, :] + s2_ref[NUM_SEGMENTS:, :]
    cnt = jnp.sum(
        cnt_ref[:, 0].reshape(_NW, NUM_SEGMENTS), axis=0)
    c = jnp.maximum(cnt, 1.0)[:, None]
    mean = s1 / c
    b = bias_ref[0]
    var = s2 / c - mean * mean + b * b
    a = lax.rsqrt(var + EPSILON)
    t_ref[:, :D_FEAT] = a
    t_ref[:, D_FEAT:] = (b - mean) * a


_WIN = 16


@functools.partial(
    pl.kernel,
    out_type=jax.ShapeDtypeStruct((N_NODES, D_FEAT), jnp.float32),
    mesh=_mesh,
    scratch_types=[
        pltpu.VMEM((_C3, D_FEAT), jnp.float32),
        pltpu.VMEM((_C3, D_FEAT), jnp.float32),
        pltpu.VMEM((_C3, D_FEAT), jnp.float32),
        pltpu.VMEM((_WIN, 2 * D_FEAT), jnp.float32),
        pltpu.VMEM((_WIN, 2 * D_FEAT), jnp.float32),
        pltpu.VMEM((_WIN, 2 * D_FEAT), jnp.float32),
        pltpu.VMEM((1, 2 * D_FEAT), jnp.float32),
        pltpu.VMEM((_MAXI3 * _C3,), jnp.int32),
        pltpu.SemaphoreType.DMA,
        pltpu.SemaphoreType.DMA,
        pltpu.SemaphoreType.DMA,
        pltpu.SemaphoreType.DMA,
        pltpu.SemaphoreType.DMA,
        pltpu.SemaphoreType.DMA,
        pltpu.SemaphoreType.DMA,
    ],
    compiler_params=_params,
)
def _k_apply(x_hbm, ids_hbm, t_hbm, out_hbm,
             xb0, xb1, xb2, tw0, tw1, tw2, trow, ivall,
             gsem0, gsem1, gsem2, osem0, osem1, osem2, isem):
    cid = lax.axis_index("c")
    sid = lax.axis_index("s")
    wid = sid * 2 + cid

    xbufs = (xb0, xb1, xb2)
    twins = (tw0, tw1, tw2)
    gsems = (gsem0, gsem1, gsem2)
    osems = (osem0, osem1, osem2)
    cols = lax.iota(jnp.int32, _L)

    def _seg_scalar(j):
        return lax.reduce_min(
            plsc.load_gather(ivall, [jnp.full((_L,), j, jnp.int32)]), (0,))

    def _win_base(i):
        return jnp.minimum(_seg_scalar(i * _C3),
                           jnp.int32(NUM_SEGMENTS - _WIN))

    for j in range(_MAXI3):
        kj = wid + j * _NW

        @pl.when(kj < _NCH3)
        def _():
            pltpu.async_copy(ids_hbm.at[pl.ds(kj * _C3, _C3)],
                             ivall.at[pl.ds(j * _C3, _C3)], isem)
    for j in range(_MAXI3):
        kj = wid + j * _NW

        @pl.when(kj < _NCH3)
        def _():
            pltpu.make_async_copy(ids_hbm.at[pl.ds(0, _C3)],
                                  ivall.at[pl.ds(0, _C3)], isem).wait()

    def _issue_pre_b(i, b):
        k = wid + i * _NW
        pltpu.async_copy(t_hbm.at[pl.ds(_win_base(i), _WIN)],
                         twins[b], gsems[b])
        pltpu.async_copy(x_hbm.at[pl.ds(k * _C3, _C3)], xbufs[b], gsems[b])

    def _wait_pre(b):
        pltpu.make_async_copy(t_hbm.at[pl.ds(0, _WIN)],
                              twins[b], gsems[b]).wait()
        pltpu.make_async_copy(x_hbm.at[pl.ds(0, _C3)],
                              xbufs[b], gsems[b]).wait()

    def _wait_out(b):
        pltpu.make_async_copy(
            xbufs[b], out_hbm.at[pl.ds(0, _C3)], osems[b]).wait()

    _issue_pre_b(0, 0)

    @pl.when(wid + _NW < _NCH3)
    def _():
        _issue_pre_b(1, 1)

    def _visit(i, b):
        k = wid + i * _NW

        @pl.when(k < _NCH3)
        def _():
            @pl.when(i >= 1)
            def _():
                _wait_out((b + 2) % 3)

            @pl.when(k + 2 * _NW < _NCH3)
            def _():
                _issue_pre_b(i + 2, (b + 2) % 3)

            _wait_pre(b)

            xb = xbufs[b]
            tw = twins[b]
            s0 = _win_base(i)
            slast = _seg_scalar(i * _C3 + _C3 - 1)

            @pl.when(slast < s0 + _WIN)
            def _():
                s0v = jnp.full((_L,), s0, jnp.int32)

                def _row(r, carry2):
                    lseg = plsc.load_gather(
                        ivall,
                        [jnp.full((_L,), i * _C3 + r, jnp.int32)]) - s0v
                    for c in range(_DV):
                        cv = cols + c * _L
                        a = plsc.load_gather(tw, [lseg, cv])
                        bv = plsc.load_gather(tw, [lseg, cv + D_FEAT])
                        v = xb[r, pl.ds(c * _L, _L)]
                        xb[r, pl.ds(c * _L, _L)] = v * a + bv
                    return carry2
                lax.fori_loop(0, _C3, _row, 0)

            @pl.when(slast >= s0 + _WIN)
            def _():
                def _row(r, carry2):
                    sr = _seg_scalar(i * _C3 + r)
                    pltpu.sync_copy(t_hbm.at[pl.ds(sr, 1)], trow)
                    for c in range(_DV):
                        a = trow[0, pl.ds(c * _L, _L)]
                        bv = trow[0, pl.ds(D_FEAT + c * _L, _L)]
                        v = xb[r, pl.ds(c * _L, _L)]
                        xb[r, pl.ds(c * _L, _L)] = v * a + bv
                    return carry2
                lax.fori_loop(0, _C3, _row, 0)

            pltpu.async_copy(xb, out_hbm.at[pl.ds(k * _C3, _C3)],
                             osems[b])

    def _outer(io, carry):
        for b3 in range(3):
            _visit(io * 3 + b3, b3)
        return carry
    lax.fori_loop(0, _MAXI3 // 3, _outer, 0)
    _visit(_MAXI3 - 1, (_MAXI3 - 1) % 3)

    @pl.when(wid < _NCH3 - (_MAXI3 - 1) * _NW)
    def _():
        _wait_out((_MAXI3 - 1) % 3)

    @pl.when(wid >= _NCH3 - (_MAXI3 - 1) * _NW)
    def _():
        _wait_out((_MAXI3 - 2) % 3)


def kernel(inputs, graph_mask, bias):
    ids = graph_mask.astype(jnp.int32)
    s1, s2, cnt = _k_reduce(inputs, ids)
    table = pl.pallas_call(
        _k_table_body,
        out_shape=jax.ShapeDtypeStruct((NUM_SEGMENTS, 2 * D_FEAT), jnp.float32),
    )(s1, s2, cnt, bias.reshape(1, D_FEAT))
    return _k_apply(inputs, ids, table)


# speedup vs baseline: 1.0775x; 1.0133x over previous
import functools

import jax
import jax.numpy as jnp
from jax import lax
from jax.experimental import pallas as pl
from jax.experimental.pallas import tpu as pltpu
from jax.experimental.pallas import tpu_sc as plsc

N_NODES = 50000
D_FEAT = 512
NUM_SEGMENTS = 1024
EPSILON = 1e-06

_L = 16
_DV = D_FEAT // _L
_NW = 32

_NH = 2
_NG = 16
_GW = 2 * _L
_HROWS = N_NODES // _NH
_C1 = 250
_NCH1 = _HROWS // _C1
_UNROLL = 10
_STRIDE = _C1 // _UNROLL

_C3 = 40
_NCH3 = N_NODES // _C3
_MAXI3 = -(-_NCH3 // _NW)

_params = pltpu.CompilerParams(use_tc_tiling_on_sc=False,
                               needs_layout_passes=False)
_mesh = plsc.VectorSubcoreMesh(core_axis_name="c", subcore_axis_name="s")


@functools.partial(
    pl.kernel,
    out_type=(
        jax.ShapeDtypeStruct((_NH * NUM_SEGMENTS, D_FEAT), jnp.float32),
        jax.ShapeDtypeStruct((_NH * NUM_SEGMENTS, D_FEAT), jnp.float32),
        jax.ShapeDtypeStruct((_NW * NUM_SEGMENTS, _L), jnp.float32),
    ),
    mesh=_mesh,
    scratch_types=[
        pltpu.VMEM((_HROWS,), jnp.int32),
        pltpu.VMEM((_C1, _GW), jnp.float32),
        pltpu.VMEM((_C1, _GW), jnp.float32),
        pltpu.VMEM((NUM_SEGMENTS, _GW), jnp.float32),
        pltpu.VMEM((NUM_SEGMENTS, _GW), jnp.float32),
        pltpu.VMEM((NUM_SEGMENTS, _L), jnp.float32),
        pltpu.SemaphoreType.DMA,
        pltpu.SemaphoreType.DMA,
    ],
    compiler_params=_params,
)
def _k_reduce(x_hbm, ids_hbm, s1_out, s2_out, cnt_out,
              iv, xb0, xb1, acc1, acc2, accc, sem0, sem1):
    cid = lax.axis_index("c")
    sid = lax.axis_index("s")
    w = sid * 2 + cid
    h = w // _NG
    g = lax.rem(w, _NG)
    f0 = g * _GW
    row0 = h * _HROWS

    zeros16 = jnp.zeros((_L,), jnp.float32)
    ones16 = jnp.ones((_L,), jnp.float32)
    cols = lax.iota(jnp.int32, _L)
    cols2 = cols + _L

    idcp = pltpu.async_copy(ids_hbm.at[pl.ds(row0, _HROWS)], iv, sem0)

    def _init(r, carry):
        acc1[r, pl.ds(0, _L)] = zeros16
        acc1[r, pl.ds(_L, _L)] = zeros16
        acc2[r, pl.ds(0, _L)] = zeros16
        acc2[r, pl.ds(_L, _L)] = zeros16
        accc[r, :] = zeros16
        return carry
    lax.fori_loop(0, NUM_SEGMENTS, _init, 0)
    idcp.wait()

    bufs = (xb0, xb1)
    sems = (sem0, sem1)

    def _issue(k, b):
        pltpu.async_copy(
            x_hbm.at[pl.ds(row0 + k * _C1, _C1), pl.ds(f0, _GW)],
            bufs[b], sems[b])

    def _wait(b):
        pltpu.make_async_copy(
            x_hbm.at[pl.ds(0, _C1), pl.ds(0, _GW)], bufs[b], sems[b]).wait()

    def _process(k, b):
        xb = bufs[b]
        rbase = k * _C1

        def _rows(jj, carry):
            for p in range(_UNROLL):
                r = p * _STRIDE + jj
                seg = plsc.load_gather(
                    iv, [jnp.full((_L,), rbase + r, jnp.int32)])
                v0 = xb[r, pl.ds(0, _L)]
                v1 = xb[r, pl.ds(_L, _L)]
                plsc.addupdate_scatter(acc1, [seg, cols], v0)
                plsc.addupdate_scatter(acc1, [seg, cols2], v1)
                plsc.addupdate_scatter(acc2, [seg, cols], v0 * v0)
                plsc.addupdate_scatter(acc2, [seg, cols2], v1 * v1)
            return carry
        lax.fori_loop(0, _STRIDE, _rows, 0)

        @pl.when(lax.rem(k, _NG) == g)
        def _():
            def _crows(jj, carry):
                for p in range(_UNROLL):
                    r = p * _STRIDE + jj
                    seg = plsc.load_gather(
                        iv, [jnp.full((_L,), rbase + r, jnp.int32)])
                    plsc.addupdate_scatter(accc, [seg, cols], ones16)
                return carry
            lax.fori_loop(0, _STRIDE, _crows, 0)

    _issue(0, 0)
    _issue(1, 1)

    def _outer(ko, carry):
        k = ko * 2
        for b in range(2):
            kk = k + b
            _wait(b)
            _process(kk, b)

            @pl.when(kk + 2 < _NCH1)
            def _():
                _issue(kk + 2, b)
        return carry
    lax.fori_loop(0, _NCH1 // 2, _outer, 0)

    obase = h * NUM_SEGMENTS
    pltpu.sync_copy(
        acc1, s1_out.at[pl.ds(obase, NUM_SEGMENTS), pl.ds(f0, _GW)])
    pltpu.sync_copy(
        acc2, s2_out.at[pl.ds(obase, NUM_SEGMENTS), pl.ds(f0, _GW)])
    pltpu.sync_copy(accc, cnt_out.at[pl.ds(w * NUM_SEGMENTS, NUM_SEGMENTS)])


def _k_table_body(s1_ref, s2_ref, cnt_ref, bias_ref, t_ref):
    s1 = s1_ref[: NUM_SEGMENTS, :] + s1_ref[NUM_SEGMENTS:, :]
    s2 = s2_ref[: NUM_SEGMENTS, :] + s2_ref[NUM_SEGMENTS:, :]
    cnt = jnp.sum(
        cnt_ref[:, 0].reshape(_NW, NUM_SEGMENTS), axis=0)
    c = jnp.maximum(cnt, 1.0)[:, None]
    mean = s1 / c
    b = bias_ref[0]
    var = s2 / c - mean * mean + b * b
    a = lax.rsqrt(var + EPSILON)
    t_ref[:, :D_FEAT] = a
    t_ref[:, D_FEAT:] = (b - mean) * a


_WIN = 16


@functools.partial(
    pl.kernel,
    out_type=jax.ShapeDtypeStruct((N_NODES, D_FEAT), jnp.float32),
    mesh=_mesh,
    scratch_types=[
        pltpu.VMEM((_C3, D_FEAT), jnp.float32),
        pltpu.VMEM((_C3, D_FEAT), jnp.float32),
        pltpu.VMEM((_C3, D_FEAT), jnp.float32),
        pltpu.VMEM((_WIN, 2 * D_FEAT), jnp.float32),
        pltpu.VMEM((_WIN, 2 * D_FEAT), jnp.float32),
        pltpu.VMEM((_WIN, 2 * D_FEAT), jnp.float32),
        pltpu.VMEM((1, 2 * D_FEAT), jnp.float32),
        pltpu.VMEM((_MAXI3 * _C3,), jnp.int32),
        pltpu.SemaphoreType.DMA,
        pltpu.SemaphoreType.DMA,
        pltpu.SemaphoreType.DMA,
        pltpu.SemaphoreType.DMA,
        pltpu.SemaphoreType.DMA,
        pltpu.SemaphoreType.DMA,
        pltpu.SemaphoreType.DMA,
    ],
    compiler_params=_params,
)
def _k_apply(x_hbm, ids_hbm, t_hbm, out_hbm,
             xb0, xb1, xb2, tw0, tw1, tw2, trow, ivall,
             gsem0, gsem1, gsem2, osem0, osem1, osem2, isem):
    cid = lax.axis_index("c")
    sid = lax.axis_index("s")
    wid = sid * 2 + cid

    xbufs = (xb0, xb1, xb2)
    twins = (tw0, tw1, tw2)
    gsems = (gsem0, gsem1, gsem2)
    osems = (osem0, osem1, osem2)
    cols = lax.iota(jnp.int32, _L)

    def _seg_scalar(j):
        return lax.reduce_min(
            plsc.load_gather(ivall, [jnp.full((_L,), j, jnp.int32)]), (0,))

    def _win_base(i):
        return jnp.minimum(_seg_scalar(i * _C3),
                           jnp.int32(NUM_SEGMENTS - _WIN))

    for j in range(_MAXI3):
        kj = wid + j * _NW

        @pl.when(kj < _NCH3)
        def _():
            pltpu.async_copy(ids_hbm.at[pl.ds(kj * _C3, _C3)],
                             ivall.at[pl.ds(j * _C3, _C3)], isem)
    for j in range(_MAXI3):
        kj = wid + j * _NW

        @pl.when(kj < _NCH3)
        def _():
            pltpu.make_async_copy(ids_hbm.at[pl.ds(0, _C3)],
                                  ivall.at[pl.ds(0, _C3)], isem).wait()

    def _issue_pre_b(i, b):
        k = wid + i * _NW
        pltpu.async_copy(t_hbm.at[pl.ds(_win_base(i), _WIN)],
                         twins[b], gsems[b])
        pltpu.async_copy(x_hbm.at[pl.ds(k * _C3, _C3)], xbufs[b], gsems[b])

    def _wait_pre(b):
        pltpu.make_async_copy(t_hbm.at[pl.ds(0, _WIN)],
                              twins[b], gsems[b]).wait()
        pltpu.make_async_copy(x_hbm.at[pl.ds(0, _C3)],
                              xbufs[b], gsems[b]).wait()

    def _wait_out(b):
        pltpu.make_async_copy(
            xbufs[b], out_hbm.at[pl.ds(0, _C3)], osems[b]).wait()

    _issue_pre_b(0, 0)

    @pl.when(wid + _NW < _NCH3)
    def _():
        _issue_pre_b(1, 1)

    def _visit(i, b):
        k = wid + i * _NW

        @pl.when(k < _NCH3)
        def _():
            @pl.when(i >= 1)
            def _():
                _wait_out((b + 2) % 3)

            @pl.when(k + 2 * _NW < _NCH3)
            def _():
                _issue_pre_b(i + 2, (b + 2) % 3)

            _wait_pre(b)

            xb = xbufs[b]
            tw = twins[b]
            s0 = _win_base(i)
            sfirst = _seg_scalar(i * _C3)
            slast = _seg_scalar(i * _C3 + _C3 - 1)

            @pl.when(slast == sfirst)
            def _():
                ls = sfirst - s0
                for c in range(_DV):
                    a = tw[ls, pl.ds(c * _L, _L)]
                    bv = tw[ls, pl.ds(D_FEAT + c * _L, _L)]

                    def _rowc(r, carry2):
                        v = xb[r, pl.ds(c * _L, _L)]
                        xb[r, pl.ds(c * _L, _L)] = v * a + bv
                        return carry2
                    lax.fori_loop(0, _C3, _rowc, 0)

            @pl.when((slast > sfirst) & (slast < s0 + _WIN))
            def _():
                s0v = jnp.full((_L,), s0, jnp.int32)

                def _row(r, carry2):
                    lseg = plsc.load_gather(
                        ivall,
                        [jnp.full((_L,), i * _C3 + r, jnp.int32)]) - s0v
                    for c in range(_DV):
                        cv = cols + c * _L
                        a = plsc.load_gather(tw, [lseg, cv])
                        bv = plsc.load_gather(tw, [lseg, cv + D_FEAT])
                        v = xb[r, pl.ds(c * _L, _L)]
                        xb[r, pl.ds(c * _L, _L)] = v * a + bv
                    return carry2
                lax.fori_loop(0, _C3, _row, 0)

            @pl.when(slast >= s0 + _WIN)
            def _():
                def _row(r, carry2):
                    sr = _seg_scalar(i * _C3 + r)
                    pltpu.sync_copy(t_hbm.at[pl.ds(sr, 1)], trow)
                    for c in range(_DV):
                        a = trow[0, pl.ds(c * _L, _L)]
                        bv = trow[0, pl.ds(D_FEAT + c * _L, _L)]
                        v = xb[r, pl.ds(c * _L, _L)]
                        xb[r, pl.ds(c * _L, _L)] = v * a + bv
                    return carry2
                lax.fori_loop(0, _C3, _row, 0)

            pltpu.async_copy(xb, out_hbm.at[pl.ds(k * _C3, _C3)],
                             osems[b])

    def _outer(io, carry):
        for b3 in range(3):
            _visit(io * 3 + b3, b3)
        return carry
    lax.fori_loop(0, _MAXI3 // 3, _outer, 0)
    _visit(_MAXI3 - 1, (_MAXI3 - 1) % 3)

    @pl.when(wid < _NCH3 - (_MAXI3 - 1) * _NW)
    def _():
        _wait_out((_MAXI3 - 1) % 3)

    @pl.when(wid >= _NCH3 - (_MAXI3 - 1) * _NW)
    def _():
        _wait_out((_MAXI3 - 2) % 3)


def kernel(inputs, graph_mask, bias):
    ids = graph_mask.astype(jnp.int32)
    s1, s2, cnt = _k_reduce(inputs, ids)
    table = pl.pallas_call(
        _k_table_body,
        out_shape=jax.ShapeDtypeStruct((NUM_SEGMENTS, 2 * D_FEAT), jnp.float32),
    )(s1, s2, cnt, bias.reshape(1, D_FEAT))
    return _k_apply(inputs, ids, table)
